# baseline (speedup 1.0000x reference)
import functools

import jax
import jax.numpy as jnp
import numpy as np
from jax import lax
from jax.experimental import pallas as pl
from jax.experimental.pallas import tpu as pltpu
from jax.experimental.pallas import tpu_sc as plsc

_N = 16384 * 200
_B = 16384

_info = plsc.get_sparse_core_info()
_NC = _info.num_cores
_NS = _info.num_subcores
_L = _info.num_lanes
_NW = _NC * _NS

_EPW = _N // _NW
_CHUNKE = 24576
_UNROLL = 8
_CHUNKS = [4096] + [24576] * 4


def _sc_gather_dot(xf, idx, label):
    mesh = plsc.VectorSubcoreMesh(core_axis_name="c", subcore_axis_name="s")

    @functools.partial(
        pl.kernel,
        mesh=mesh,
        out_type=jax.ShapeDtypeStruct((_NW, _L), jnp.float32),
        scratch_types=[
            pltpu.VMEM((_B,), jnp.float32),
            pltpu.VMEM((_CHUNKE,), jnp.float32),
            pltpu.VMEM((_CHUNKE,), jnp.float32),
            pltpu.VMEM((_CHUNKE,), jnp.int32),
            pltpu.VMEM((_CHUNKE,), jnp.int32),
            pltpu.VMEM((_L,), jnp.float32),
            pltpu.SemaphoreType.DMA,
            pltpu.SemaphoreType.DMA,
            pltpu.SemaphoreType.DMA,
        ],
        compiler_params=pltpu.CompilerParams(needs_layout_passes=False),
    )
    def body(x_hbm, idx_hbm, label_hbm, out_hbm,
             label_v, x0_v, x1_v, i0_v, i1_v, acc_v, sem0, sem1, sem_l):
        wid = lax.axis_index("s") * _NC + lax.axis_index("c")
        ebase = wid * _EPW
        xbufs = (x0_v, x1_v)
        ibufs = (i0_v, i1_v)
        sems = (sem0, sem1)
        offs = [sum(_CHUNKS[:c]) for c in range(len(_CHUNKS))]

        def start(c):
            src = pl.ds(ebase + offs[c], _CHUNKS[c])
            b = c % 2
            return (pltpu.async_copy(
                        x_hbm.at[src], xbufs[b].at[pl.ds(0, _CHUNKS[c])],
                        sems[b]),
                    pltpu.async_copy(
                        idx_hbm.at[src], ibufs[b].at[pl.ds(0, _CHUNKS[c])],
                        sems[b]))

        pending = start(0)
        label_h = pltpu.async_copy(label_hbm, label_v, sem_l)
        accs = (jnp.zeros((_L,), jnp.float32),) * 4
        for c in range(len(_CHUNKS)):
            for h in pending:
                h.wait()
            if c == 0:
                label_h.wait()
            if c + 1 < len(_CHUNKS):
                pending = start(c + 1)
            x_v = xbufs[c % 2]
            idx_v = ibufs[c % 2]

            def step(r, accs):
                accs = list(accs)
                for u in range(_UNROLL):
                    o = r * _L * _UNROLL + u * _L
                    xv = x_v[pl.ds(o, _L)]
                    iv = idx_v[pl.ds(o, _L)]
                    g = plsc.load_gather(label_v, [iv])
                    accs[u % 4] = accs[u % 4] + xv * g
                return tuple(accs)

            accs = lax.fori_loop(0, _CHUNKS[c] // (_L * _UNROLL), step, accs)

        acc_v[...] = (accs[0] + accs[1]) + (accs[2] + accs[3])
        pltpu.sync_copy(acc_v, out_hbm.at[wid])

    return body(xf, idx, label)


_TC_GRID = 8
_TC_BLK = _N // _TC_GRID


def _tc_dense_body(x_ref, o_ref):
    v = x_ref[...].reshape(_TC_BLK // 128, 128)
    val = jnp.maximum(v, 0.0) + jnp.log1p(jnp.exp(-jnp.abs(v)))
    s = jnp.sum(val, axis=0, keepdims=True)

    @pl.when(pl.program_id(0) == 0)
    def _init():
        o_ref[...] = s

    @pl.when(pl.program_id(0) != 0)
    def _acc():
        o_ref[...] += s


def _tc_dense_sum(xf):
    return pl.pallas_call(
        _tc_dense_body,
        grid=(_TC_GRID,),
        in_specs=[pl.BlockSpec((_TC_BLK,), lambda i: (i,))],
        out_specs=pl.BlockSpec((1, 128), lambda i: (0, 0)),
        out_shape=jax.ShapeDtypeStruct((1, 128), jnp.float32),
    )(xf)


def kernel(X, batch_idx, label):
    xf = X.reshape(_N)
    sc_parts = _sc_gather_dot(xf, batch_idx.astype(jnp.int32), label)
    tc_parts = _tc_dense_sum(xf)
    total = jnp.sum(tc_parts) - jnp.sum(sc_parts)
    return total * np.float32(1.0 / _N)


# speedup vs baseline: 1.0160x; 1.0160x over previous
import functools

import jax
import jax.numpy as jnp
import numpy as np
from jax import lax
from jax.experimental import pallas as pl
from jax.experimental.pallas import tpu as pltpu
from jax.experimental.pallas import tpu_sc as plsc

_N = 16384 * 200
_B = 16384

_info = plsc.get_sparse_core_info()
_NC = _info.num_cores
_NS = _info.num_subcores
_L = _info.num_lanes
_NW = _NC * _NS

_EPW = _N // _NW
_CHUNKE = 25600
_UNROLL = 8
_CHUNKS = [25600] * 4


def _sc_gather_dot(xf, idx, label):
    mesh = plsc.VectorSubcoreMesh(core_axis_name="c", subcore_axis_name="s")

    @functools.partial(
        pl.kernel,
        mesh=mesh,
        out_type=jax.ShapeDtypeStruct((_NW, _L), jnp.float32),
        scratch_types=[
            pltpu.VMEM((_B,), jnp.float32),
            pltpu.VMEM((_CHUNKE,), jnp.float32),
            pltpu.VMEM((_CHUNKE,), jnp.float32),
            pltpu.VMEM((_CHUNKE,), jnp.int32),
            pltpu.VMEM((_CHUNKE,), jnp.int32),
            pltpu.VMEM((_L,), jnp.float32),
            pltpu.SemaphoreType.DMA,
            pltpu.SemaphoreType.DMA,
            pltpu.SemaphoreType.DMA,
        ],
        compiler_params=pltpu.CompilerParams(needs_layout_passes=False),
    )
    def body(x_hbm, idx_hbm, label_hbm, out_hbm,
             label_v, x0_v, x1_v, i0_v, i1_v, acc_v, sem0, sem1, sem_l):
        wid = lax.axis_index("s") * _NC + lax.axis_index("c")
        ebase = wid * _EPW
        xbufs = (x0_v, x1_v)
        ibufs = (i0_v, i1_v)
        sems = (sem0, sem1)
        offs = [sum(_CHUNKS[:c]) for c in range(len(_CHUNKS))]

        def start(c):
            src = pl.ds(ebase + offs[c], _CHUNKS[c])
            b = c % 2
            return (pltpu.async_copy(
                        x_hbm.at[src], xbufs[b].at[pl.ds(0, _CHUNKS[c])],
                        sems[b]),
                    pltpu.async_copy(
                        idx_hbm.at[src], ibufs[b].at[pl.ds(0, _CHUNKS[c])],
                        sems[b]))

        pending = start(0)
        label_h = pltpu.async_copy(label_hbm, label_v, sem_l)
        accs = (jnp.zeros((_L,), jnp.float32),) * 4
        for c in range(len(_CHUNKS)):
            for h in pending:
                h.wait()
            if c == 0:
                label_h.wait()
            if c + 1 < len(_CHUNKS):
                pending = start(c + 1)
            x_v = xbufs[c % 2]
            idx_v = ibufs[c % 2]

            def step(r, accs):
                accs = list(accs)
                for u in range(_UNROLL):
                    o = r * _L * _UNROLL + u * _L
                    xv = x_v[pl.ds(o, _L)]
                    iv = idx_v[pl.ds(o, _L)]
                    g = plsc.load_gather(label_v, [iv])
                    accs[u % 4] = accs[u % 4] + xv * g
                return tuple(accs)

            accs = lax.fori_loop(0, _CHUNKS[c] // (_L * _UNROLL), step, accs)

        acc_v[...] = (accs[0] + accs[1]) + (accs[2] + accs[3])
        pltpu.sync_copy(acc_v, out_hbm.at[wid])

    return body(xf, idx, label)


_TC_GRID = 8
_TC_BLK = _N // _TC_GRID


def _tc_dense_body(x_ref, o_ref):
    v = x_ref[...].reshape(_TC_BLK // 128, 128)
    val = jnp.maximum(v, 0.0) + jnp.log1p(jnp.exp(-jnp.abs(v)))
    s = jnp.sum(val, axis=0, keepdims=True)

    @pl.when(pl.program_id(0) == 0)
    def _init():
        o_ref[...] = s

    @pl.when(pl.program_id(0) != 0)
    def _acc():
        o_ref[...] += s


def _tc_dense_sum(xf):
    return pl.pallas_call(
        _tc_dense_body,
        grid=(_TC_GRID,),
        in_specs=[pl.BlockSpec((_TC_BLK,), lambda i: (i,))],
        out_specs=pl.BlockSpec((1, 128), lambda i: (0, 0)),
        out_shape=jax.ShapeDtypeStruct((1, 128), jnp.float32),
    )(xf)


def kernel(X, batch_idx, label):
    xf = X.reshape(_N)
    sc_parts = _sc_gather_dot(xf, batch_idx.astype(jnp.int32), label)
    tc_parts = _tc_dense_sum(xf)
    total = jnp.sum(tc_parts) - jnp.sum(sc_parts)
    return total * np.float32(1.0 / _N)
